# baseline (speedup 1.0000x reference)
import functools
import math

import jax
import jax.numpy as jnp
from jax import lax
from jax.experimental import pallas as pl
from jax.experimental.pallas import tpu as pltpu
from jax.experimental.pallas import tpu_sc as plsc

VOCAB = 1000000
EMBED_DIM = 64
BATCH = 4096
HIST = 200

_NC = 2
_NS = 16
_NW = _NC * _NS
_BW = BATCH // _NW
_PER_W = _BW * HIST
_WIDE = 2 * EMBED_DIM
_L = 16
_BG = _BW // _L
_SCALE = math.sqrt(EMBED_DIM)


def _emb_body(idx_hbm, tab_hbm, out_hbm,
              idx_all, dummy_idx,
              rows0, rows1, tile0, tile1,
              gsem0, gsem1, osem0, osem1):
    wid = lax.axis_index("s") * _NC + lax.axis_index("c")

    pltpu.sync_copy(idx_hbm.at[pl.ds(wid * _PER_W, _PER_W)], idx_all)

    iota16 = lax.iota(jnp.int32, 16)
    bufs = ((rows0, tile0, gsem0, osem0),
            (rows1, tile1, gsem1, osem1))

    def start_gather(h, rows, gsem):
        for g in range(_BG):
            flat = (iota16 + g * _L) * HIST + h
            v16 = plsc.load_gather(idx_all, [flat])
            pltpu.async_copy(
                tab_hbm.at[v16], rows.at[pl.ds(g * _L, _L)], gsem)

    def wait_gather(rows, gsem):
        pltpu.make_async_copy(tab_hbm.at[dummy_idx], rows, gsem).wait()

    def transpose_scale(rows, tile):
        def body(d, carry):
            col16 = jnp.full((16,), 0, jnp.int32) + d
            for g in range(_BG):
                row16 = iota16 + g * _L
                v = plsc.load_gather(rows, [row16, col16])
                tile[d, pl.ds(g * _L, _L)] = v * _SCALE
            return carry

        lax.fori_loop(0, EMBED_DIM, body, 0, unroll=2)

    def start_out(h, tile, osem):
        for dg in range(8):
            pltpu.async_copy(
                tile.at[pl.ds(dg * 8, 8)], out_hbm.at[h, dg, wid], osem)

    def wait_out(tile, osem):
        for dg in range(8):
            pltpu.make_async_copy(
                tile.at[pl.ds(dg * 8, 8)], out_hbm.at[0, dg, wid], osem).wait()

    start_gather(0, rows0, gsem0)

    def step(j2, carry):
        for b in range(2):
            rows_b, tile_b, gsem_b, osem_b = bufs[b]
            rows_o, tile_o, gsem_o, osem_o = bufs[1 - b]
            cur = j2 * 2 + b

            @pl.when((cur >= 1) & (cur + 1 < HIST))
            def _():
                wait_out(tile_o, osem_o)

            @pl.when(cur + 1 < HIST)
            def _():
                start_gather(cur + 1, rows_o, gsem_o)

            wait_gather(rows_b, gsem_b)
            transpose_scale(rows_b, tile_b)
            start_out(cur, tile_b, osem_b)
        return carry

    lax.fori_loop(0, HIST // 2, step, 0)

    wait_out(tile0, osem0)
    wait_out(tile1, osem1)


_mesh = plsc.VectorSubcoreMesh(core_axis_name="c", subcore_axis_name="s")

_emb = functools.partial(
    pl.kernel,
    mesh=_mesh,
    out_type=jax.ShapeDtypeStruct((HIST, 8, _NW, 8, 128), jnp.float32),
    scratch_types=[
        pltpu.VMEM((_PER_W,), jnp.int32),
        pltpu.VMEM((_BW,), jnp.int32),
        pltpu.VMEM((_BW, _WIDE), jnp.float32),
        pltpu.VMEM((_BW, _WIDE), jnp.float32),
        pltpu.VMEM((EMBED_DIM, _BW), jnp.float32),
        pltpu.VMEM((EMBED_DIM, _BW), jnp.float32),
        pltpu.SemaphoreType.DMA,
        pltpu.SemaphoreType.DMA,
        pltpu.SemaphoreType.DMA,
        pltpu.SemaphoreType.DMA,
    ],
    compiler_params=pltpu.CompilerParams(
        use_tc_tiling_on_sc=False, needs_layout_passes=False),
)(_emb_body)


def kernel(x, table):
    flat = x.reshape(-1).astype(jnp.int32)
    wide = jnp.pad(table, ((0, 0), (0, EMBED_DIM)))
    out5 = _emb(flat, wide)
    return out5.transpose(2, 4, 0, 1, 3).reshape(BATCH, HIST, EMBED_DIM)
